# Initial kernel scaffold; baseline (speedup 1.0000x reference)
import jax
import jax.numpy as jnp
from jax.experimental import pallas as pl


def kernel(x, W_enc, b_enc, W_dec, b_dec):
    raise NotImplementedError("write your pallas kernel here")


# speedup vs baseline: 2.5373x; 2.5373x over previous
import jax
import jax.numpy as jnp
from jax.experimental import pallas as pl
from jax.experimental.pallas import tpu as pltpu

_B, _D, _F, _K = 32, 2048, 65536, 128
_TILE = 1024
_T = _F // _TILE
_CH = 4096


def _sortable_u32(v):
    b = jax.lax.bitcast_convert_type(v, jnp.uint32)
    neg = b >= jnp.uint32(0x80000000)
    return jnp.where(neg, ~b, b | jnp.uint32(0x80000000))


def _unsort_f32(k):
    pos = k >= jnp.uint32(0x80000000)
    b = jnp.where(pos, k ^ jnp.uint32(0x80000000), ~k)
    return jax.lax.bitcast_convert_type(b, jnp.float32)


def _body(x_ref, be_ref, bd_ref, We_ref, Wd_ref, dec_ref, enc_ref, key_s, thr_s):
    i = pl.program_id(0)

    @pl.when(i < _T)
    def _encode():
        emb = jax.lax.dot_general(
            x_ref[...], We_ref[...], (((1,), (1,)), ((), ())),
            preferred_element_type=jnp.float32)
        emb = emb + be_ref[...]
        key_s[:, pl.ds(i * _TILE, _TILE)] = _sortable_u32(emb)

    @pl.when(i == _T)
    def _threshold():
        def count_ge(mid):
            def chunk(c, acc):
                kk = key_s[:, pl.ds(c * _CH, _CH)]
                return acc + jnp.sum((kk >= mid).astype(jnp.int32), axis=1,
                                     keepdims=True)
            return jax.lax.fori_loop(0, _F // _CH, chunk,
                                     jnp.zeros((_B, 1), jnp.int32))

        def step(_, carry):
            lo, hi = carry
            mid = lo + (hi - lo + jnp.uint32(1)) // jnp.uint32(2)
            ok = count_ge(mid) >= _K
            lo = jnp.where(ok, mid, lo)
            hi = jnp.where(ok, hi, mid - jnp.uint32(1))
            return lo, hi

        lo0 = jnp.zeros((_B, 1), jnp.uint32)
        hi0 = jnp.full((_B, 1), jnp.uint32(0xFFFFFFFE))
        lo, _ = jax.lax.fori_loop(0, 32, step, (lo0, hi0))
        thr_s[...] = lo

    @pl.when(i >= _T)
    def _decode():
        j = i - _T
        kk = key_s[:, pl.ds(j * _TILE, _TILE)]
        thr = thr_s[...]
        emb = _unsort_f32(kk)
        enc = jnp.where(kk >= thr, emb, jnp.float32(0.0))
        enc_ref[...] = enc
        contrib = jax.lax.dot_general(
            enc, Wd_ref[...], (((1,), (1,)), ((), ())),
            preferred_element_type=jnp.float32)

        @pl.when(j == 0)
        def _():
            dec_ref[...] = contrib + bd_ref[...]

        @pl.when(j > 0)
        def _():
            dec_ref[...] = dec_ref[...] + contrib


def kernel(x, W_enc, b_enc, W_dec, b_dec):
    be2 = b_enc.reshape(1, _F)
    bd2 = b_dec.reshape(1, _D)
    dec, enc = pl.pallas_call(
        _body,
        grid=(2 * _T,),
        in_specs=[
            pl.BlockSpec((_B, _D), lambda i: (0, 0)),
            pl.BlockSpec((1, _TILE), lambda i: (0, jnp.minimum(i, _T - 1))),
            pl.BlockSpec((1, _D), lambda i: (0, 0)),
            pl.BlockSpec((_TILE, _D), lambda i: (jnp.minimum(i, _T - 1), 0)),
            pl.BlockSpec((_D, _TILE), lambda i: (0, jnp.maximum(i - _T, 0))),
        ],
        out_specs=[
            pl.BlockSpec((_B, _D), lambda i: (0, 0)),
            pl.BlockSpec((_B, _TILE), lambda i: (0, jnp.maximum(i - _T, 0))),
        ],
        out_shape=[
            jax.ShapeDtypeStruct((_B, _D), jnp.float32),
            jax.ShapeDtypeStruct((_B, _F), jnp.float32),
        ],
        scratch_shapes=[
            pltpu.VMEM((_B, _F), jnp.uint32),
            pltpu.VMEM((_B, 1), jnp.uint32),
        ],
    )(x, be2, bd2, W_enc, W_dec)
    return (dec, enc, x)


# speedup vs baseline: 2.6139x; 1.0302x over previous
import jax
import jax.numpy as jnp
from jax.experimental import pallas as pl
from jax.experimental.pallas import tpu as pltpu

_B, _D, _F, _K = 32, 2048, 65536, 128
_TE = 1024
_NTE = _F // _TE
_TD = 2048
_NTD = _F // _TD
_CH = 4096


def _sortable_u32(v):
    b = jax.lax.bitcast_convert_type(v, jnp.uint32)
    neg = b >= jnp.uint32(0x80000000)
    return jnp.where(neg, ~b, b | jnp.uint32(0x80000000))


def _unsort_f32(k):
    pos = k >= jnp.uint32(0x80000000)
    b = jnp.where(pos, k ^ jnp.uint32(0x80000000), ~k)
    return jax.lax.bitcast_convert_type(b, jnp.float32)


def _body(x_ref, be_ref, bd_ref, We_ref, Wd_ref, dec_ref, enc_ref, key_s, thr_s):
    i = pl.program_id(0)

    @pl.when(i < _NTE)
    def _encode():
        emb = jax.lax.dot_general(
            x_ref[...], We_ref[...], (((1,), (1,)), ((), ())),
            preferred_element_type=jnp.float32)
        emb = emb + be_ref[...]
        key_s[:, pl.ds(i * _TE, _TE)] = _sortable_u32(emb)

    @pl.when(i == _NTE)
    def _threshold():
        def count_ge(mid):
            def chunk(c, acc):
                kk = key_s[:, pl.ds(c * _CH, _CH)]
                return acc + jnp.sum((kk >= mid).astype(jnp.int32), axis=1,
                                     keepdims=True)
            return jax.lax.fori_loop(0, _F // _CH, chunk,
                                     jnp.zeros((_B, 1), jnp.int32))

        def step(_, carry):
            lo, hi = carry
            mid = lo + (hi - lo + jnp.uint32(1)) // jnp.uint32(2)
            ok = count_ge(mid) >= _K
            lo = jnp.where(ok, mid, lo)
            hi = jnp.where(ok, hi, mid - jnp.uint32(1))
            return lo, hi

        lo0 = jnp.zeros((_B, 1), jnp.uint32)
        hi0 = jnp.full((_B, 1), jnp.uint32(0xFFFFFFFE))
        lo, _ = jax.lax.fori_loop(0, 32, step, (lo0, hi0))
        thr_s[...] = lo

    @pl.when(i >= _NTE)
    def _decode():
        j = i - _NTE
        kk = key_s[:, pl.ds(j * _TD, _TD)]
        thr = thr_s[...]
        emb = _unsort_f32(kk)
        enc = jnp.where(kk >= thr, emb, jnp.float32(0.0))
        enc_ref[...] = enc
        contrib = jax.lax.dot_general(
            enc, Wd_ref[...], (((1,), (1,)), ((), ())),
            preferred_element_type=jnp.float32)

        @pl.when(j == 0)
        def _():
            dec_ref[...] = contrib + bd_ref[...]

        @pl.when(j > 0)
        def _():
            dec_ref[...] = dec_ref[...] + contrib


def kernel(x, W_enc, b_enc, W_dec, b_dec):
    be2 = b_enc.reshape(1, _F)
    bd2 = b_dec.reshape(1, _D)
    dec, enc = pl.pallas_call(
        _body,
        grid=(_NTE + _NTD,),
        in_specs=[
            pl.BlockSpec((_B, _D), lambda i: (0, 0)),
            pl.BlockSpec((1, _TE), lambda i: (0, jnp.minimum(i, _NTE - 1))),
            pl.BlockSpec((1, _D), lambda i: (0, 0)),
            pl.BlockSpec((_TE, _D), lambda i: (jnp.minimum(i, _NTE - 1), 0)),
            pl.BlockSpec((_D, _TD), lambda i: (0, jnp.maximum(i - _NTE, 0))),
        ],
        out_specs=[
            pl.BlockSpec((_B, _D), lambda i: (0, 0)),
            pl.BlockSpec((_B, _TD), lambda i: (0, jnp.maximum(i - _NTE, 0))),
        ],
        out_shape=[
            jax.ShapeDtypeStruct((_B, _D), jnp.float32),
            jax.ShapeDtypeStruct((_B, _F), jnp.float32),
        ],
        scratch_shapes=[
            pltpu.VMEM((_B, _F), jnp.uint32),
            pltpu.VMEM((_B, 1), jnp.uint32),
        ],
    )(x, be2, bd2, W_enc, W_dec)
    return (dec, enc, x)
